# Initial kernel scaffold; baseline (speedup 1.0000x reference)
import jax
import jax.numpy as jnp
from jax.experimental import pallas as pl


def kernel(feat_vec, labels, centers, weights):
    raise NotImplementedError("write your pallas kernel here")


# speedup vs baseline: 2.7025x; 2.7025x over previous
import functools

import jax
import jax.numpy as jnp
from jax import lax
from jax.experimental import pallas as pl
from jax.experimental.pallas import tpu as pltpu
from jax.experimental.pallas import tpu_sc as plsc

_K = 1000
_D = 128
_B = 16384
_ALPHA = 0.1
_KPAD = 1024

_NC = 2
_NS = 16
_NW = _NC * _NS
_BPW = _B // _NW
_CH = 128
_NCH = _BPW // _CH
_L = 16


def _sc_body(feat_hbm, labels_hbm, centers_hbm, weights_hbm, out_hbm,
             labels_v, wv, crow0, crow1, feat0, feat1, stage,
             sem_w, sem_c0, sem_c1, sem_f0, sem_f1):
    cid = lax.axis_index("c")
    sid = lax.axis_index("s")
    wid = sid * _NC + cid
    base = wid * _BPW

    pltpu.sync_copy(labels_hbm.at[pl.ds(base, _BPW)], labels_v)

    wcopies = [
        pltpu.async_copy(weights_hbm.at[labels_v.at[pl.ds(i * _CH, _CH)]],
                         wv.at[pl.ds(i * _CH, _CH)], sem_w)
        for i in range(_NCH)
    ]

    crows = (crow0, crow1)
    feats = (feat0, feat1)
    semc = (sem_c0, sem_c1)
    semf = (sem_f0, sem_f1)

    def start(ci):
        buf = ci % 2
        return (
            pltpu.async_copy(centers_hbm.at[labels_v.at[pl.ds(ci * _CH, _CH)]],
                             crows[buf], semc[buf]),
            pltpu.async_copy(feat_hbm.at[pl.ds(base + ci * _CH, _CH)],
                             feats[buf], semf[buf]),
        )

    copies = start(0)
    for w in wcopies:
        w.wait()

    total = jnp.zeros((_L,), jnp.float32)
    for ci in range(_NCH):
        buf = ci % 2
        nxt = start(ci + 1) if ci + 1 < _NCH else None
        copies[0].wait()
        copies[1].wait()

        crow_ref = crows[buf]
        feat_ref = feats[buf]
        off = ci * _CH

        def group_body(g, tot, _crow=crow_ref, _feat=feat_ref, _off=off):
            t = tot
            wvec = wv[pl.ds(_off + g * _L, _L)]
            for l in range(_L):
                b = g * _L + l
                accs = [jnp.zeros((_L,), jnp.float32) for _ in range(2)]
                for j in range(_D // _L):
                    f = _feat[b, pl.ds(j * _L, _L)]
                    c = _crow[b, pl.ds(j * _L, _L)]
                    dfc = f - c
                    accs[j % 2] = accs[j % 2] + dfc * dfc
                t = t + (accs[0] + accs[1]) * wvec[l]
            return t

        total = lax.fori_loop(0, _CH // _L, group_body, total)
        copies = nxt

    stage[...] = total
    pltpu.sync_copy(stage, out_hbm.at[wid])


@functools.partial(
    pl.kernel,
    mesh=plsc.VectorSubcoreMesh(core_axis_name="c", subcore_axis_name="s"),
    out_type=jax.ShapeDtypeStruct((_NW, _L), jnp.float32),
    scratch_types=[
        pltpu.VMEM((_BPW,), jnp.int32),
        pltpu.VMEM((_BPW,), jnp.float32),
        pltpu.VMEM((_CH, _D), jnp.float32),
        pltpu.VMEM((_CH, _D), jnp.float32),
        pltpu.VMEM((_CH, _D), jnp.float32),
        pltpu.VMEM((_CH, _D), jnp.float32),
        pltpu.VMEM((_L,), jnp.float32),
        pltpu.SemaphoreType.DMA,
        pltpu.SemaphoreType.DMA,
        pltpu.SemaphoreType.DMA,
        pltpu.SemaphoreType.DMA,
        pltpu.SemaphoreType.DMA,
    ],
)
def _sc_term1(feat_hbm, labels_hbm, centers_hbm, weights_hbm, out_hbm,
              *scratch):
    _sc_body(feat_hbm, labels_hbm, centers_hbm, weights_hbm, out_hbm,
             *scratch)


def _t2_body(c_ref, out_ref):
    c = c_ref[...]
    cc = c * c
    q_col = jnp.sum(cc, axis=1, keepdims=True)
    ones = jnp.ones((1, _D), jnp.float32)
    q_row = lax.dot_general(ones, cc, (((1,), (1,)), ((), ())),
                            preferred_element_type=jnp.float32)
    g = lax.dot_general(c, c, (((1,), (1,)), ((), ())),
                        preferred_element_type=jnp.float32)
    d = jnp.maximum(q_col + q_row - 2.0 * g, 0.0)
    ii = lax.broadcasted_iota(jnp.int32, (_KPAD, _KPAD), 0)
    jj = lax.broadcasted_iota(jnp.int32, (_KPAD, _KPAD), 1)
    keep = (jj > ii) & (jj < _K)
    out_ref[0, 0] = jnp.sum(jnp.where(keep, d, 0.0))


def _t2_sum(centers_padded):
    return pl.pallas_call(
        _t2_body,
        out_shape=jax.ShapeDtypeStruct((1, 1), jnp.float32),
        out_specs=pl.BlockSpec(memory_space=pltpu.SMEM),
    )(centers_padded)


def kernel(feat_vec, labels, centers, weights):
    labels = labels.astype(jnp.int32)
    partials = _sc_term1(feat_vec, labels, centers, weights)
    cpad = jnp.pad(centers, ((0, _KPAD - _K), (0, 0)))
    t2 = _t2_sum(cpad)[0, 0]
    t1 = 0.5 * jnp.sum(partials) / _B
    return t1 - _ALPHA * t2 / float(_K * _K)


# speedup vs baseline: 3.8577x; 1.4275x over previous
import functools

import jax
import jax.numpy as jnp
from jax import lax
from jax.experimental import pallas as pl
from jax.experimental.pallas import tpu as pltpu
from jax.experimental.pallas import tpu_sc as plsc

_K = 1000
_D = 128
_B = 16384
_ALPHA = 0.1
_KPAD = 1024

_NC = 2
_NS = 16
_NW = _NC * _NS
_BPW = _B // _NW
_CH = 128
_NCH = _BPW // _CH
_L = 16


def _sc_body(feat_hbm, labels_hbm, centers_hbm, weights_hbm, out_hbm,
             labels_v, wv, wexp, crow0, crow1, feat0, feat1, stage,
             sem_w, sem_c0, sem_c1, sem_f0, sem_f1):
    cid = lax.axis_index("c")
    sid = lax.axis_index("s")
    wid = sid * _NC + cid
    base = wid * _BPW

    pltpu.sync_copy(labels_hbm.at[pl.ds(base, _BPW)], labels_v)

    wcopies = [
        pltpu.async_copy(weights_hbm.at[labels_v.at[pl.ds(i * _CH, _CH)]],
                         wv.at[pl.ds(i * _CH, _CH)], sem_w)
        for i in range(_NCH)
    ]

    crows = (crow0, crow1)
    feats = (feat0, feat1)
    semc = (sem_c0, sem_c1)
    semf = (sem_f0, sem_f1)

    def start(ci):
        buf = ci % 2
        return (
            pltpu.async_copy(centers_hbm.at[labels_v.at[pl.ds(ci * _CH, _CH)]],
                             crows[buf], semc[buf]),
            pltpu.async_copy(feat_hbm.at[pl.ds(base + ci * _CH, _CH)],
                             feats[buf], semf[buf]),
        )

    copies = start(0)
    for w in wcopies:
        w.wait()

    def wexp_body(g, z):
        wvec = wv[pl.ds(g * _L, _L)]
        for l in range(_L):
            wexp[pl.ds((g * _L + l) * _L, _L)] = jnp.broadcast_to(wvec[l],
                                                                  (_L,))
        return z

    lax.fori_loop(0, _BPW // _L, wexp_body, 0)

    total = tuple(jnp.zeros((_L,), jnp.float32) for _ in range(4))
    for ci in range(_NCH):
        buf = ci % 2
        nxt = start(ci + 1) if ci + 1 < _NCH else None
        copies[0].wait()
        copies[1].wait()

        crow_ref = crows[buf]
        feat_ref = feats[buf]
        off = ci * _CH

        def samp_body(b, carry, _crow=crow_ref, _feat=feat_ref, _off=off):
            ts = list(carry)
            wb = wexp[pl.ds((_off + b) * _L, _L)]
            for j in range(_D // _L):
                f = _feat[b, pl.ds(j * _L, _L)]
                c = _crow[b, pl.ds(j * _L, _L)]
                dfc = f - c
                ts[j % 4] = ts[j % 4] + (wb * dfc) * dfc
            return tuple(ts)

        total = lax.fori_loop(0, _CH, samp_body, total, unroll=2)
        copies = nxt

    stage[...] = (total[0] + total[1]) + (total[2] + total[3])
    pltpu.sync_copy(stage, out_hbm.at[wid])


@functools.partial(
    pl.kernel,
    mesh=plsc.VectorSubcoreMesh(core_axis_name="c", subcore_axis_name="s"),
    out_type=jax.ShapeDtypeStruct((_NW, _L), jnp.float32),
    scratch_types=[
        pltpu.VMEM((_BPW,), jnp.int32),
        pltpu.VMEM((_BPW,), jnp.float32),
        pltpu.VMEM((_BPW * _L,), jnp.float32),
        pltpu.VMEM((_CH, _D), jnp.float32),
        pltpu.VMEM((_CH, _D), jnp.float32),
        pltpu.VMEM((_CH, _D), jnp.float32),
        pltpu.VMEM((_CH, _D), jnp.float32),
        pltpu.VMEM((_L,), jnp.float32),
        pltpu.SemaphoreType.DMA,
        pltpu.SemaphoreType.DMA,
        pltpu.SemaphoreType.DMA,
        pltpu.SemaphoreType.DMA,
        pltpu.SemaphoreType.DMA,
    ],
)
def _sc_term1(feat_hbm, labels_hbm, centers_hbm, weights_hbm, out_hbm,
              *scratch):
    _sc_body(feat_hbm, labels_hbm, centers_hbm, weights_hbm, out_hbm,
             *scratch)


def _t2_body(c_ref, out_ref):
    c = c_ref[...]
    cc = c * c
    q_col = jnp.sum(cc, axis=1, keepdims=True)
    ones = jnp.ones((1, _D), jnp.float32)
    q_row = lax.dot_general(ones, cc, (((1,), (1,)), ((), ())),
                            preferred_element_type=jnp.float32)
    g = lax.dot_general(c, c, (((1,), (1,)), ((), ())),
                        preferred_element_type=jnp.float32)
    d = jnp.maximum(q_col + q_row - 2.0 * g, 0.0)
    ii = lax.broadcasted_iota(jnp.int32, (_KPAD, _KPAD), 0)
    jj = lax.broadcasted_iota(jnp.int32, (_KPAD, _KPAD), 1)
    keep = (jj > ii) & (jj < _K)
    out_ref[0, 0] = jnp.sum(jnp.where(keep, d, 0.0))


def _t2_sum(centers_padded):
    return pl.pallas_call(
        _t2_body,
        out_shape=jax.ShapeDtypeStruct((1, 1), jnp.float32),
        out_specs=pl.BlockSpec(memory_space=pltpu.SMEM),
    )(centers_padded)


def kernel(feat_vec, labels, centers, weights):
    labels = labels.astype(jnp.int32)
    partials = _sc_term1(feat_vec, labels, centers, weights)
    cpad = jnp.pad(centers, ((0, _KPAD - _K), (0, 0)))
    t2 = _t2_sum(cpad)[0, 0]
    t1 = 0.5 * jnp.sum(partials) / _B
    return t1 - _ALPHA * t2 / float(_K * _K)
